# Initial kernel scaffold; baseline (speedup 1.0000x reference)
import jax
import jax.numpy as jnp
from jax.experimental import pallas as pl


def kernel(factor_potentials, prev_factor_beliefs, prev_var_beliefs, edge_factor_idx, edge_var_idx, edge_dim, W3_1, b3_1, W3_2, b3_2, W4_1, b4_1, W4_2, b4_2):
    raise NotImplementedError("write your pallas kernel here")


# speedup vs baseline: 20.5888x; 20.5888x over previous
import functools

import jax
import jax.numpy as jnp
from jax import lax
from jax.experimental import pallas as pl
from jax.experimental.pallas import tpu as pltpu
from jax.experimental.pallas import tpu_sc as plsc

V = 10000
F = 160000
E = 320000
C = 8
FSS = C * C

NC = 2
NS = 16
NW = NC * NS

FH = F // NC
DUMP = FH


def _vector_mesh():
    return plsc.VectorSubcoreMesh(core_axis_name="c", subcore_axis_name="s")


_SC_PARAMS = pltpu.CompilerParams(use_tc_tiling_on_sc=False)


def _sc_gather(table, idx, chunk):
    n, d = table.shape
    e = idx.shape[0]
    per_w = e // NW
    n_chunks = per_w // chunk

    @functools.partial(
        pl.kernel,
        out_type=jax.ShapeDtypeStruct((e, d), jnp.float32),
        mesh=_vector_mesh(),
        compiler_params=_SC_PARAMS,
        scratch_types=[
            pltpu.VMEM((chunk,), jnp.int32),
            pltpu.VMEM((chunk, d), jnp.float32),
        ],
    )
    def k(table_hbm, idx_hbm, out_hbm, idx_v, rows_v):
        wid = lax.axis_index("s") * NC + lax.axis_index("c")
        base = wid * per_w

        @pl.loop(0, n_chunks)
        def _(i):
            off = base + i * chunk
            pltpu.sync_copy(idx_hbm.at[pl.ds(off, chunk)], idx_v)
            pltpu.sync_copy(table_hbm.at[idx_v], rows_v)
            pltpu.sync_copy(rows_v, out_hbm.at[pl.ds(off, chunk)])

    return k(table, idx)


def _sc_var_scatter(vals16, idx, zeros_hbm_src):
    chunk = 1000
    per_sc = E // NC
    per_w = per_sc // NS
    n_chunks = per_w // chunk

    @functools.partial(
        pl.kernel,
        out_type=jax.ShapeDtypeStruct((NC, V, 16), jnp.float32),
        mesh=_vector_mesh(),
        compiler_params=_SC_PARAMS,
        scratch_types=[
            pltpu.VMEM((chunk,), jnp.int32),
            pltpu.VMEM((chunk, 16), jnp.float32),
            pltpu.VMEM_SHARED((V, 16), jnp.float32),
        ],
    )
    def k(vals_hbm, idx_hbm, zeros_hbm, out_hbm, idx_v, vals_v, acc_sh):
        cid = lax.axis_index("c")
        sid = lax.axis_index("s")

        @pl.when(sid == 0)
        def _():
            pltpu.sync_copy(zeros_hbm.at[pl.ds(0, V)], acc_sh)

        plsc.subcore_barrier()
        base = cid * per_sc + sid * per_w

        @pl.loop(0, n_chunks)
        def _(i):
            off = base + i * chunk
            pltpu.sync_copy(idx_hbm.at[pl.ds(off, chunk)], idx_v)
            pltpu.sync_copy(vals_hbm.at[pl.ds(off, chunk)], vals_v)
            pltpu.sync_copy(vals_v, acc_sh.at[idx_v], add=True)

        plsc.subcore_barrier()

        @pl.when(sid < 8)
        def _():
            rows = V // 8
            pltpu.sync_copy(acc_sh.at[pl.ds(sid * rows, rows)],
                            out_hbm.at[cid].at[pl.ds(sid * rows, rows)])

    return k(vals16, idx, zeros_hbm_src)


def _sc_factor_scatter(avals, idx, zeros_hbm_src):
    chunk = 800
    per_w = E // NS
    n_chunks = per_w // chunk

    @functools.partial(
        pl.kernel,
        out_type=jax.ShapeDtypeStruct((F, 16), jnp.float32),
        mesh=_vector_mesh(),
        compiler_params=_SC_PARAMS,
        scratch_types=[
            pltpu.VMEM((chunk,), jnp.int32),
            pltpu.VMEM((chunk, 16), jnp.float32),
            pltpu.VMEM_SHARED((FH + 8, 16), jnp.float32),
        ],
    )
    def k(vals_hbm, idx_hbm, zeros_hbm, out_hbm, idx_v, vals_v, acc_sh):
        cid = lax.axis_index("c")
        sid = lax.axis_index("s")

        @pl.when(sid == 0)
        def _():
            pltpu.sync_copy(zeros_hbm, acc_sh)

        plsc.subcore_barrier()
        base = sid * per_w
        lo = cid * FH

        @pl.loop(0, n_chunks)
        def _(i):
            off = base + i * chunk
            pltpu.sync_copy(idx_hbm.at[pl.ds(off, chunk)], idx_v)
            pltpu.sync_copy(vals_hbm.at[pl.ds(off, chunk)], vals_v)

            @pl.loop(0, chunk // 16)
            def _(j):
                sl = pl.ds(j * 16, 16)
                x = idx_v[sl] - lo
                ok = (x >= 0) & (x < FH)
                idx_v[sl] = jnp.where(ok, x, DUMP)

            pltpu.sync_copy(vals_v, acc_sh.at[idx_v], add=True)

        plsc.subcore_barrier()
        rows = FH // NS
        pltpu.sync_copy(acc_sh.at[pl.ds(sid * rows, rows)],
                        out_hbm.at[pl.ds(cid * FH + sid * rows, rows)])

    return k(avals, idx, zeros_hbm_src)


_BLK3 = 512
_BLK4 = 640


def _tc_mlp3(fb_edges, edim2, w1, b1, w2, b2):
    def body(fb_ref, d_ref, w1_ref, b1_ref, w2_ref, b2_ref, o_ref):
        x = jnp.exp(fb_ref[...])
        h = jnp.maximum(
            jnp.dot(x, w1_ref[...], preferred_element_type=jnp.float32)
            + b1_ref[...], 0.0)
        z = jnp.maximum(
            jnp.dot(h, w2_ref[...], preferred_element_type=jnp.float32)
            + b2_ref[...], 0.0) + 1e-19
        cc = lax.broadcasted_iota(jnp.int32, (FSS, C), 0)
        ii = lax.broadcasted_iota(jnp.int32, (FSS, C), 1)
        g0 = (cc // C == ii).astype(jnp.float32)
        g1 = (cc % C == ii).astype(jnp.float32)
        s0 = jnp.dot(z, g0, preferred_element_type=jnp.float32)
        s1 = jnp.dot(z, g1, preferred_element_type=jnp.float32)
        ftov = jnp.log(jnp.where(d_ref[...] == 0, s0, s1))
        o_ref[...] = jnp.concatenate([ftov, jnp.zeros_like(ftov)], axis=1)

    return pl.pallas_call(
        body,
        grid=(E // _BLK3,),
        in_specs=[
            pl.BlockSpec((_BLK3, FSS), lambda i: (i, 0)),
            pl.BlockSpec((_BLK3, 1), lambda i: (i, 0)),
            pl.BlockSpec((FSS, 2 * FSS), lambda i: (0, 0)),
            pl.BlockSpec((1, 2 * FSS), lambda i: (0, 0)),
            pl.BlockSpec((2 * FSS, FSS), lambda i: (0, 0)),
            pl.BlockSpec((1, FSS), lambda i: (0, 0)),
        ],
        out_specs=pl.BlockSpec((_BLK3, 16), lambda i: (i, 0)),
        out_shape=jax.ShapeDtypeStruct((E, 16), jnp.float32),
    )(fb_edges, edim2, w1, b1.reshape(1, -1), w2, b2.reshape(1, -1))


def _tc_damp(pvb16, p0, p1):
    def body(p_ref, a_ref, b_ref, o_ref):
        o_ref[...] = 0.5 * p_ref[...] + 0.5 * (a_ref[...] + b_ref[...])

    return pl.pallas_call(
        body,
        out_shape=jax.ShapeDtypeStruct((V, 16), jnp.float32),
    )(pvb16, p0, p1)


def _tc_avals(nvb_e, ftov16, edim2):
    def body(n_ref, f_ref, d_ref, o_ref):
        vt = n_ref[...][:, 0:C] - f_ref[...][:, 0:C]
        is0 = d_ref[...] == 0
        a0 = jnp.where(is0, vt, 0.0)
        a1 = jnp.where(is0, 0.0, vt)
        o_ref[...] = jnp.concatenate([a0, a1], axis=1)

    return pl.pallas_call(
        body,
        grid=(E // _BLK3,),
        in_specs=[
            pl.BlockSpec((_BLK3, 16), lambda i: (i, 0)),
            pl.BlockSpec((_BLK3, 16), lambda i: (i, 0)),
            pl.BlockSpec((_BLK3, 1), lambda i: (i, 0)),
        ],
        out_specs=pl.BlockSpec((_BLK3, 16), lambda i: (i, 0)),
        out_shape=jax.ShapeDtypeStruct((E, 16), jnp.float32),
    )(nvb_e, ftov16, edim2)


def _tc_mlp4(pot, av, w1, b1, w2, b2):
    def body(p_ref, a_ref, w1_ref, b1_ref, w2_ref, b2_ref, o_ref):
        a = a_ref[...]
        ii = lax.broadcasted_iota(jnp.int32, (C, FSS), 0)
        cc = lax.broadcasted_iota(jnp.int32, (C, FSS), 1)
        r0 = (cc // C == ii).astype(jnp.float32)
        r1 = (cc % C == ii).astype(jnp.float32)
        x = (p_ref[...]
             + jnp.dot(a[:, 0:C], r0, preferred_element_type=jnp.float32)
             + jnp.dot(a[:, C:2 * C], r1, preferred_element_type=jnp.float32))
        e = jnp.exp(x)
        h = jnp.maximum(
            jnp.dot(e, w1_ref[...], preferred_element_type=jnp.float32)
            + b1_ref[...], 0.0)
        z = jnp.maximum(
            jnp.dot(h, w2_ref[...], preferred_element_type=jnp.float32)
            + b2_ref[...], 0.0) + 1e-19
        o_ref[...] = jnp.log(z)

    return pl.pallas_call(
        body,
        grid=(F // _BLK4,),
        in_specs=[
            pl.BlockSpec((_BLK4, FSS), lambda i: (i, 0)),
            pl.BlockSpec((_BLK4, 16), lambda i: (i, 0)),
            pl.BlockSpec((FSS, 2 * FSS), lambda i: (0, 0)),
            pl.BlockSpec((1, 2 * FSS), lambda i: (0, 0)),
            pl.BlockSpec((2 * FSS, FSS), lambda i: (0, 0)),
            pl.BlockSpec((1, FSS), lambda i: (0, 0)),
        ],
        out_specs=pl.BlockSpec((_BLK4, FSS), lambda i: (i, 0)),
        out_shape=jax.ShapeDtypeStruct((F, FSS), jnp.float32),
    )(pot, av, w1, b1.reshape(1, -1), w2, b2.reshape(1, -1))


def kernel(factor_potentials, prev_factor_beliefs, prev_var_beliefs,
           edge_factor_idx, edge_var_idx, edge_dim,
           W3_1, b3_1, W3_2, b3_2, W4_1, b4_1, W4_2, b4_2):
    efi = edge_factor_idx.astype(jnp.int32)
    evi = edge_var_idx.astype(jnp.int32)
    edim2 = edge_dim.astype(jnp.int32).reshape(E, 1)
    pfb = prev_factor_beliefs.reshape(F, FSS)
    pot = factor_potentials.reshape(F, FSS)
    pvb16 = jnp.pad(prev_var_beliefs, ((0, 0), (0, 16 - C)))
    zeros16 = jnp.zeros((FH + 8, 16), jnp.float32)

    fb_edges = _sc_gather(pfb, efi, chunk=1000)
    ftov16 = _tc_mlp3(fb_edges, edim2, W3_1, b3_1, W3_2, b3_2)
    vparts = _sc_var_scatter(ftov16, evi, zeros16)
    nvb16 = _tc_damp(pvb16, vparts[0], vparts[1])
    nvb_e = _sc_gather(nvb16, evi, chunk=2000)
    avals = _tc_avals(nvb_e, ftov16, edim2)
    av = _sc_factor_scatter(avals, efi, zeros16)
    nf = _tc_mlp4(pot, av, W4_1, b4_1, W4_2, b4_2)

    return nvb16[:, :C], nf.reshape(F, C, C)


# speedup vs baseline: 28.6947x; 1.3937x over previous
import functools

import jax
import jax.numpy as jnp
from jax import lax
from jax.experimental import pallas as pl
from jax.experimental.pallas import tpu as pltpu
from jax.experimental.pallas import tpu_sc as plsc

V = 10000
F = 160000
E = 320000
C = 8
FSS = C * C

NC = 2
NS = 16
NW = NC * NS

FH = F // NC
DUMP = FH


def _vector_mesh():
    return plsc.VectorSubcoreMesh(core_axis_name="c", subcore_axis_name="s")


_SC_PARAMS = pltpu.CompilerParams(use_tc_tiling_on_sc=False)


def _sc_gather(table, idx, chunk):
    n, d = table.shape
    e = idx.shape[0]
    per_w = e // NW
    n_chunks = per_w // chunk

    @functools.partial(
        pl.kernel,
        out_type=jax.ShapeDtypeStruct((e, d), jnp.float32),
        mesh=_vector_mesh(),
        compiler_params=_SC_PARAMS,
        scratch_types=[
            pltpu.VMEM((chunk,), jnp.int32),
            pltpu.VMEM((chunk, d), jnp.float32),
        ],
    )
    def k(table_hbm, idx_hbm, out_hbm, idx_v, rows_v):
        wid = lax.axis_index("s") * NC + lax.axis_index("c")
        base = wid * per_w

        @pl.loop(0, n_chunks)
        def _(i):
            off = base + i * chunk
            pltpu.sync_copy(idx_hbm.at[pl.ds(off, chunk)], idx_v)
            pltpu.sync_copy(table_hbm.at[idx_v], rows_v)
            pltpu.sync_copy(rows_v, out_hbm.at[pl.ds(off, chunk)])

    return k(table, idx)


def _sc_var_scatter(vals16, idx, zeros_hbm_src):
    chunk = 1000
    per_sc = E // NC
    per_w = per_sc // NS
    n_chunks = per_w // chunk

    @functools.partial(
        pl.kernel,
        out_type=jax.ShapeDtypeStruct((NC, V, 16), jnp.float32),
        mesh=_vector_mesh(),
        compiler_params=_SC_PARAMS,
        scratch_types=[
            pltpu.VMEM((chunk,), jnp.int32),
            pltpu.VMEM((chunk, 16), jnp.float32),
            pltpu.VMEM_SHARED((V, 16), jnp.float32),
        ],
    )
    def k(vals_hbm, idx_hbm, zeros_hbm, out_hbm, idx_v, vals_v, acc_sh):
        cid = lax.axis_index("c")
        sid = lax.axis_index("s")

        @pl.when(sid == 0)
        def _():
            pltpu.sync_copy(zeros_hbm.at[pl.ds(0, V)], acc_sh)

        plsc.subcore_barrier()
        base = cid * per_sc + sid * per_w

        @pl.loop(0, n_chunks)
        def _(i):
            off = base + i * chunk
            pltpu.sync_copy(idx_hbm.at[pl.ds(off, chunk)], idx_v)
            pltpu.sync_copy(vals_hbm.at[pl.ds(off, chunk)], vals_v)
            pltpu.sync_copy(vals_v, acc_sh.at[idx_v], add=True)

        plsc.subcore_barrier()

        @pl.when(sid < 8)
        def _():
            rows = V // 8
            pltpu.sync_copy(acc_sh.at[pl.ds(sid * rows, rows)],
                            out_hbm.at[cid].at[pl.ds(sid * rows, rows)])

    return k(vals16, idx, zeros_hbm_src)


def _sc_factor_scatter(avals, idx, zeros_hbm_src):
    chunk = 800
    per_w = E // NS
    n_chunks = per_w // chunk

    @functools.partial(
        pl.kernel,
        out_type=jax.ShapeDtypeStruct((F, 16), jnp.float32),
        mesh=_vector_mesh(),
        compiler_params=_SC_PARAMS,
        scratch_types=[
            pltpu.VMEM((chunk,), jnp.int32),
            pltpu.VMEM((chunk, 16), jnp.float32),
            pltpu.VMEM_SHARED((FH + 8, 16), jnp.float32),
        ],
    )
    def k(vals_hbm, idx_hbm, zeros_hbm, out_hbm, idx_v, vals_v, acc_sh):
        cid = lax.axis_index("c")
        sid = lax.axis_index("s")

        @pl.when(sid == 0)
        def _():
            pltpu.sync_copy(zeros_hbm, acc_sh)

        plsc.subcore_barrier()
        base = sid * per_w
        lo = cid * FH

        @pl.loop(0, n_chunks)
        def _(i):
            off = base + i * chunk
            pltpu.sync_copy(idx_hbm.at[pl.ds(off, chunk)], idx_v)
            pltpu.sync_copy(vals_hbm.at[pl.ds(off, chunk)], vals_v)

            @pl.loop(0, chunk // 16)
            def _(j):
                sl = pl.ds(j * 16, 16)
                x = idx_v[sl] - lo
                ok = (x >= 0) & (x < FH)
                idx_v[sl] = jnp.where(ok, x, DUMP)

            pltpu.sync_copy(vals_v, acc_sh.at[idx_v], add=True)

        plsc.subcore_barrier()
        rows = FH // NS
        pltpu.sync_copy(acc_sh.at[pl.ds(sid * rows, rows)],
                        out_hbm.at[pl.ds(cid * FH + sid * rows, rows)])

    return k(avals, idx, zeros_hbm_src)


_BLK3 = 1280
_BLKA = 6400
_BLK4 = 1280


def _tc_mlp3(fb_edges, edim2, w1, b1, w2, b2):
    def body(fb_ref, d_ref, w1_ref, b1_ref, w2_ref, b2_ref, o_ref):
        x = jnp.exp(fb_ref[...])
        h = jnp.maximum(
            jnp.dot(x, w1_ref[...], preferred_element_type=jnp.float32)
            + b1_ref[...], 0.0)
        z = jnp.maximum(
            jnp.dot(h, w2_ref[...], preferred_element_type=jnp.float32)
            + b2_ref[...], 0.0) + 1e-19
        cc = lax.broadcasted_iota(jnp.int32, (FSS, C), 0)
        ii = lax.broadcasted_iota(jnp.int32, (FSS, C), 1)
        g0 = (cc // C == ii).astype(jnp.float32)
        g1 = (cc % C == ii).astype(jnp.float32)
        s0 = jnp.dot(z, g0, preferred_element_type=jnp.float32)
        s1 = jnp.dot(z, g1, preferred_element_type=jnp.float32)
        d = (d_ref[...] == 0)
        ftov = jnp.log(jnp.where(d, s0, s1))
        o_ref[...] = jnp.concatenate(
            [ftov, jnp.broadcast_to(d, ftov.shape).astype(jnp.float32)], axis=1)

    return pl.pallas_call(
        body,
        grid=(E // _BLK3,),
        in_specs=[
            pl.BlockSpec((_BLK3, FSS), lambda i: (i, 0)),
            pl.BlockSpec((_BLK3, 1), lambda i: (i, 0)),
            pl.BlockSpec((FSS, 2 * FSS), lambda i: (0, 0)),
            pl.BlockSpec((1, 2 * FSS), lambda i: (0, 0)),
            pl.BlockSpec((2 * FSS, FSS), lambda i: (0, 0)),
            pl.BlockSpec((1, FSS), lambda i: (0, 0)),
        ],
        out_specs=pl.BlockSpec((_BLK3, 16), lambda i: (i, 0)),
        out_shape=jax.ShapeDtypeStruct((E, 16), jnp.float32),
    )(fb_edges, edim2, w1, b1.reshape(1, -1), w2, b2.reshape(1, -1))


def _tc_damp(pvb16, p0, p1):
    def body(p_ref, a_ref, b_ref, o_ref):
        o_ref[...] = 0.5 * p_ref[...] + 0.5 * (a_ref[...] + b_ref[...])

    return pl.pallas_call(
        body,
        out_shape=jax.ShapeDtypeStruct((V, 16), jnp.float32),
    )(pvb16, p0, p1)


def _tc_avals(nvb_e, ftov16):
    def body(n_ref, f_ref, o_ref):
        f = f_ref[...]
        vt = n_ref[...][:, 0:C] - f[:, 0:C]
        m0 = f[:, C:2 * C]
        a0 = vt * m0
        a1 = vt - a0
        o_ref[...] = jnp.concatenate([a0, a1], axis=1)

    return pl.pallas_call(
        body,
        grid=(E // _BLKA,),
        in_specs=[
            pl.BlockSpec((_BLKA, 16), lambda i: (i, 0)),
            pl.BlockSpec((_BLKA, 16), lambda i: (i, 0)),
        ],
        out_specs=pl.BlockSpec((_BLKA, 16), lambda i: (i, 0)),
        out_shape=jax.ShapeDtypeStruct((E, 16), jnp.float32),
    )(nvb_e, ftov16)


def _tc_mlp4(pot, av, w1, b1, w2, b2):
    def body(p_ref, a_ref, w1_ref, b1_ref, w2_ref, b2_ref, o_ref):
        a = a_ref[...]
        ii = lax.broadcasted_iota(jnp.int32, (16, FSS), 0)
        cc = lax.broadcasted_iota(jnp.int32, (16, FSS), 1)
        rt = jnp.where(ii < C, (cc // C == ii).astype(jnp.float32),
                       (cc % C == ii - C).astype(jnp.float32))
        x = (p_ref[...]
             + jnp.dot(a, rt, preferred_element_type=jnp.float32))
        e = jnp.exp(x)
        h = jnp.maximum(
            jnp.dot(e, w1_ref[...], preferred_element_type=jnp.float32)
            + b1_ref[...], 0.0)
        z = jnp.maximum(
            jnp.dot(h, w2_ref[...], preferred_element_type=jnp.float32)
            + b2_ref[...], 0.0) + 1e-19
        o_ref[...] = jnp.log(z)

    return pl.pallas_call(
        body,
        grid=(F // _BLK4,),
        in_specs=[
            pl.BlockSpec((_BLK4, FSS), lambda i: (i, 0)),
            pl.BlockSpec((_BLK4, 16), lambda i: (i, 0)),
            pl.BlockSpec((FSS, 2 * FSS), lambda i: (0, 0)),
            pl.BlockSpec((1, 2 * FSS), lambda i: (0, 0)),
            pl.BlockSpec((2 * FSS, FSS), lambda i: (0, 0)),
            pl.BlockSpec((1, FSS), lambda i: (0, 0)),
        ],
        out_specs=pl.BlockSpec((_BLK4, FSS), lambda i: (i, 0)),
        out_shape=jax.ShapeDtypeStruct((F, FSS), jnp.float32),
    )(pot, av, w1, b1.reshape(1, -1), w2, b2.reshape(1, -1))


def kernel(factor_potentials, prev_factor_beliefs, prev_var_beliefs,
           edge_factor_idx, edge_var_idx, edge_dim,
           W3_1, b3_1, W3_2, b3_2, W4_1, b4_1, W4_2, b4_2):
    efi = edge_factor_idx.astype(jnp.int32)
    evi = edge_var_idx.astype(jnp.int32)
    edim2 = edge_dim.astype(jnp.int32).reshape(E, 1)
    pfb = prev_factor_beliefs.reshape(F, FSS)
    pot = factor_potentials.reshape(F, FSS)
    pvb16 = jnp.pad(prev_var_beliefs, ((0, 0), (0, 16 - C)))
    zeros16 = jnp.zeros((FH + 8, 16), jnp.float32)

    fb_edges = _sc_gather(pfb, efi, chunk=1000)
    ftov16 = _tc_mlp3(fb_edges, edim2, W3_1, b3_1, W3_2, b3_2)
    vparts = _sc_var_scatter(ftov16, evi, zeros16)
    nvb16 = _tc_damp(pvb16, vparts[0], vparts[1])
    nvb_e = _sc_gather(nvb16, evi, chunk=2000)
    avals = _tc_avals(nvb_e, ftov16)
    av = _sc_factor_scatter(avals, efi, zeros16)
    nf = _tc_mlp4(pot, av, W4_1, b4_1, W4_2, b4_2)

    return nvb16[:, :C], nf.reshape(F, C, C)


# speedup vs baseline: 30.3961x; 1.0593x over previous
import functools

import jax
import jax.numpy as jnp
from jax import lax
from jax.experimental import pallas as pl
from jax.experimental.pallas import tpu as pltpu
from jax.experimental.pallas import tpu_sc as plsc

V = 10000
F = 160000
E = 320000
C = 8
FSS = C * C

NC = 2
NS = 16
NW = NC * NS

FH = F // NC
DUMP = FH


def _vector_mesh():
    return plsc.VectorSubcoreMesh(core_axis_name="c", subcore_axis_name="s")


_SC_PARAMS = pltpu.CompilerParams(use_tc_tiling_on_sc=False)


def _sc_gather(table, idx, chunk):
    n, d = table.shape
    e = idx.shape[0]
    per_w = e // NW
    n_chunks = per_w // chunk

    @functools.partial(
        pl.kernel,
        out_type=jax.ShapeDtypeStruct((e, d), jnp.float32),
        mesh=_vector_mesh(),
        compiler_params=_SC_PARAMS,
        scratch_types=[
            pltpu.VMEM((chunk,), jnp.int32),
            pltpu.VMEM((chunk, d), jnp.float32),
        ],
    )
    def k(table_hbm, idx_hbm, out_hbm, idx_v, rows_v):
        wid = lax.axis_index("s") * NC + lax.axis_index("c")
        base = wid * per_w

        @pl.loop(0, n_chunks)
        def _(i):
            off = base + i * chunk
            pltpu.sync_copy(idx_hbm.at[pl.ds(off, chunk)], idx_v)
            pltpu.sync_copy(table_hbm.at[idx_v], rows_v)
            pltpu.sync_copy(rows_v, out_hbm.at[pl.ds(off, chunk)])

    return k(table, idx)


def _sc_var_scatter(vals16, idx, zeros_hbm_src):
    chunk = 1000
    per_sc = E // NC
    per_w = per_sc // NS
    n_chunks = per_w // chunk

    @functools.partial(
        pl.kernel,
        out_type=jax.ShapeDtypeStruct((NC, V, 16), jnp.float32),
        mesh=_vector_mesh(),
        compiler_params=_SC_PARAMS,
        scratch_types=[
            pltpu.VMEM((chunk,), jnp.int32),
            pltpu.VMEM((chunk, 16), jnp.float32),
            pltpu.VMEM_SHARED((V, 16), jnp.float32),
        ],
    )
    def k(vals_hbm, idx_hbm, zeros_hbm, out_hbm, idx_v, vals_v, acc_sh):
        cid = lax.axis_index("c")
        sid = lax.axis_index("s")

        @pl.when(sid == 0)
        def _():
            pltpu.sync_copy(zeros_hbm.at[pl.ds(0, V)], acc_sh)

        plsc.subcore_barrier()
        base = cid * per_sc + sid * per_w

        @pl.loop(0, n_chunks)
        def _(i):
            off = base + i * chunk
            pltpu.sync_copy(idx_hbm.at[pl.ds(off, chunk)], idx_v)
            pltpu.sync_copy(vals_hbm.at[pl.ds(off, chunk)], vals_v)
            pltpu.sync_copy(vals_v, acc_sh.at[idx_v], add=True)

        plsc.subcore_barrier()

        @pl.when(sid < 8)
        def _():
            rows = V // 8
            pltpu.sync_copy(acc_sh.at[pl.ds(sid * rows, rows)],
                            out_hbm.at[cid].at[pl.ds(sid * rows, rows)])

    return k(vals16, idx, zeros_hbm_src)


def _sc_factor_scatter(avals, idx, zeros_hbm_src):
    chunk = 800
    per_w = E // NS
    n_chunks = per_w // chunk

    @functools.partial(
        pl.kernel,
        out_type=jax.ShapeDtypeStruct((F, 16), jnp.float32),
        mesh=_vector_mesh(),
        compiler_params=_SC_PARAMS,
        scratch_types=[
            pltpu.VMEM((chunk,), jnp.int32),
            pltpu.VMEM((chunk, 16), jnp.float32),
            pltpu.VMEM_SHARED((FH + 8, 16), jnp.float32),
        ],
    )
    def k(vals_hbm, idx_hbm, zeros_hbm, out_hbm, idx_v, vals_v, acc_sh):
        cid = lax.axis_index("c")
        sid = lax.axis_index("s")

        @pl.when(sid == 0)
        def _():
            pltpu.sync_copy(zeros_hbm, acc_sh)

        plsc.subcore_barrier()
        base = sid * per_w
        lo = cid * FH

        @pl.loop(0, n_chunks)
        def _(i):
            off = base + i * chunk
            pltpu.sync_copy(idx_hbm.at[pl.ds(off, chunk)], idx_v)
            pltpu.sync_copy(vals_hbm.at[pl.ds(off, chunk)], vals_v)

            @pl.loop(0, chunk // 16)
            def _(j):
                sl = pl.ds(j * 16, 16)
                x = idx_v[sl] - lo
                ok = (x >= 0) & (x < FH)
                idx_v[sl] = jnp.where(ok, x, DUMP)

            pltpu.sync_copy(vals_v, acc_sh.at[idx_v], add=True)

        plsc.subcore_barrier()
        rows = FH // NS
        pltpu.sync_copy(acc_sh.at[pl.ds(sid * rows, rows)],
                        out_hbm.at[pl.ds(cid * FH + sid * rows, rows)])

    return k(avals, idx, zeros_hbm_src)


_BLK3 = 1280
_BLKA = 6400
_BLK4 = 1280


def _tc_mlp3(fb_edges, edim2, w1, b1, w2, b2):
    def body(fb_ref, d_ref, w1_ref, b1_ref, w2_ref, b2_ref, o_ref):
        x = jnp.exp(fb_ref[...][:, 0:FSS])
        h = jnp.maximum(
            jnp.dot(x, w1_ref[...], preferred_element_type=jnp.float32)
            + b1_ref[...], 0.0)
        z = jnp.maximum(
            jnp.dot(h, w2_ref[...], preferred_element_type=jnp.float32)
            + b2_ref[...], 0.0) + 1e-19
        cc = lax.broadcasted_iota(jnp.int32, (FSS, C), 0)
        ii = lax.broadcasted_iota(jnp.int32, (FSS, C), 1)
        g0 = (cc // C == ii).astype(jnp.float32)
        g1 = (cc % C == ii).astype(jnp.float32)
        s0 = jnp.dot(z, g0, preferred_element_type=jnp.float32)
        s1 = jnp.dot(z, g1, preferred_element_type=jnp.float32)
        d = (d_ref[...] == 0)
        ftov = jnp.log(jnp.where(d, s0, s1))
        o_ref[...] = jnp.concatenate(
            [ftov, jnp.broadcast_to(d, ftov.shape).astype(jnp.float32)], axis=1)

    return pl.pallas_call(
        body,
        grid=(E // _BLK3,),
        in_specs=[
            pl.BlockSpec((_BLK3, 128), lambda i: (i, 0)),
            pl.BlockSpec((_BLK3, 1), lambda i: (i, 0)),
            pl.BlockSpec((FSS, 2 * FSS), lambda i: (0, 0)),
            pl.BlockSpec((1, 2 * FSS), lambda i: (0, 0)),
            pl.BlockSpec((2 * FSS, FSS), lambda i: (0, 0)),
            pl.BlockSpec((1, FSS), lambda i: (0, 0)),
        ],
        out_specs=pl.BlockSpec((_BLK3, 16), lambda i: (i, 0)),
        out_shape=jax.ShapeDtypeStruct((E, 16), jnp.float32),
    )(fb_edges, edim2, w1, b1.reshape(1, -1), w2, b2.reshape(1, -1))


def _tc_damp(pvb16, p0, p1):
    def body(p_ref, a_ref, b_ref, o_ref):
        o_ref[...] = 0.5 * p_ref[...] + 0.5 * (a_ref[...] + b_ref[...])

    return pl.pallas_call(
        body,
        out_shape=jax.ShapeDtypeStruct((V, 16), jnp.float32),
    )(pvb16, p0, p1)


def _tc_avals(nvb_e, ftov16):
    def body(n_ref, f_ref, o_ref):
        f = f_ref[...]
        vt = n_ref[...][:, 0:C] - f[:, 0:C]
        m0 = f[:, C:2 * C]
        a0 = vt * m0
        a1 = vt - a0
        o_ref[...] = jnp.concatenate([a0, a1], axis=1)

    return pl.pallas_call(
        body,
        grid=(E // _BLKA,),
        in_specs=[
            pl.BlockSpec((_BLKA, 16), lambda i: (i, 0)),
            pl.BlockSpec((_BLKA, 16), lambda i: (i, 0)),
        ],
        out_specs=pl.BlockSpec((_BLKA, 16), lambda i: (i, 0)),
        out_shape=jax.ShapeDtypeStruct((E, 16), jnp.float32),
    )(nvb_e, ftov16)


def _tc_mlp4(pot, av, w1, b1, w2, b2):
    def body(p_ref, a_ref, w1_ref, b1_ref, w2_ref, b2_ref, o_ref):
        a = a_ref[...]
        ii = lax.broadcasted_iota(jnp.int32, (16, FSS), 0)
        cc = lax.broadcasted_iota(jnp.int32, (16, FSS), 1)
        rt = jnp.where(ii < C, (cc // C == ii).astype(jnp.float32),
                       (cc % C == ii - C).astype(jnp.float32))
        x = (p_ref[...]
             + jnp.dot(a, rt, preferred_element_type=jnp.float32))
        e = jnp.exp(x)
        h = jnp.maximum(
            jnp.dot(e, w1_ref[...], preferred_element_type=jnp.float32)
            + b1_ref[...], 0.0)
        z = jnp.maximum(
            jnp.dot(h, w2_ref[...], preferred_element_type=jnp.float32)
            + b2_ref[...], 0.0) + 1e-19
        o_ref[...] = jnp.log(z)

    return pl.pallas_call(
        body,
        grid=(F // _BLK4,),
        in_specs=[
            pl.BlockSpec((_BLK4, FSS), lambda i: (i, 0)),
            pl.BlockSpec((_BLK4, 16), lambda i: (i, 0)),
            pl.BlockSpec((FSS, 2 * FSS), lambda i: (0, 0)),
            pl.BlockSpec((1, 2 * FSS), lambda i: (0, 0)),
            pl.BlockSpec((2 * FSS, FSS), lambda i: (0, 0)),
            pl.BlockSpec((1, FSS), lambda i: (0, 0)),
        ],
        out_specs=pl.BlockSpec((_BLK4, FSS), lambda i: (i, 0)),
        out_shape=jax.ShapeDtypeStruct((F, FSS), jnp.float32),
    )(pot, av, w1, b1.reshape(1, -1), w2, b2.reshape(1, -1))


def kernel(factor_potentials, prev_factor_beliefs, prev_var_beliefs,
           edge_factor_idx, edge_var_idx, edge_dim,
           W3_1, b3_1, W3_2, b3_2, W4_1, b4_1, W4_2, b4_2):
    efi = edge_factor_idx.astype(jnp.int32)
    evi = edge_var_idx.astype(jnp.int32)
    edim2 = edge_dim.astype(jnp.int32).reshape(E, 1)
    pfb = jnp.pad(prev_factor_beliefs.reshape(F, FSS), ((0, 0), (0, 64)))
    pot = factor_potentials.reshape(F, FSS)
    pvb16 = jnp.pad(prev_var_beliefs, ((0, 0), (0, 16 - C)))
    zeros16 = jnp.zeros((FH + 8, 16), jnp.float32)

    fb_edges = _sc_gather(pfb, efi, chunk=1000)
    ftov16 = _tc_mlp3(fb_edges, edim2, W3_1, b3_1, W3_2, b3_2)
    vparts = _sc_var_scatter(ftov16, evi, zeros16)
    nvb16 = _tc_damp(pvb16, vparts[0], vparts[1])
    nvb_e = _sc_gather(nvb16, evi, chunk=2000)
    avals = _tc_avals(nvb_e, ftov16)
    av = _sc_factor_scatter(avals, efi, zeros16)
    nf = _tc_mlp4(pot, av, W4_1, b4_1, W4_2, b4_2)

    return nvb16[:, :C], nf.reshape(F, C, C)
